# baseline (speedup 1.0000x reference)
import functools

import jax
import jax.numpy as jnp
import numpy as np
from jax.experimental import pallas as pl
from jax.experimental.pallas import tpu as pltpu

H = 1024
E = 16
I = 512
IS = 2048
N_GROUP = 4
GROUP_SIZE = E // N_GROUP
TOPK_GROUP = 2
TOP_K = 8
ROUTED_SCALE = 2.5

T = 2048
TBLK = 256


def _rank_desc(v):
    B, N = v.shape
    idx = jax.lax.broadcasted_iota(jnp.int32, (B, N), 1)
    rank = jnp.zeros((B, N), jnp.float32)
    for j in range(N):
        colv = v[:, j:j + 1]
        beats = jnp.logical_or(colv > v,
                               jnp.logical_and(colv == v, j < idx))
        rank = rank + jnp.where(beats, 1.0, 0.0)
    return rank


def _router_kernel(s_ref, bias_ref, cmb_ref):
    s = s_ref[...]
    sc = s + bias_ref[...]

    col = [sc[:, j:j + 1] for j in range(E)]

    top2 = []
    for g in range(N_GROUP):
        a, b, c, d = col[4 * g], col[4 * g + 1], col[4 * g + 2], col[4 * g + 3]
        hi1, lo1 = jnp.maximum(a, b), jnp.minimum(a, b)
        hi2, lo2 = jnp.maximum(c, d), jnp.minimum(c, d)
        top2.append(jnp.maximum(jnp.maximum(hi1 + hi2, hi1 + lo1), hi2 + lo2))

    lane = jax.lax.broadcasted_iota(jnp.int32, (TBLK, E), 1)
    zero = jnp.zeros((TBLK, E), jnp.float32)
    esel = zero
    for g in range(N_GROUP):
        grank = 0
        for g2 in range(N_GROUP):
            if g2 == g:
                continue
            beats = jnp.logical_or(
                top2[g2] > top2[g],
                jnp.logical_and(top2[g2] == top2[g], g2 < g))
            grank = grank + jnp.where(beats, 1, 0)
        gsel = grank < TOPK_GROUP
        gmask = jnp.logical_and(lane >= 4 * g, lane < 4 * (g + 1))
        esel = esel + jnp.where(jnp.logical_and(gsel, gmask), 1.0, 0.0)

    scores_for_choice = jnp.where(esel > 0.5, sc, 0.0)

    erank = _rank_desc(scores_for_choice)
    sel = erank < TOP_K

    tw = jnp.where(sel, s, 0.0)
    denom = jnp.sum(tw, axis=1, keepdims=True) + 1e-20
    cmb_ref[...] = tw * (ROUTED_SCALE / denom)


def _moe_kernel(cmb_ref, x_ref, wu_ref, wd_ref, wus_ref, wds_ref, out_ref,
                xbf_ref):
    i = pl.program_id(0)
    routed = i < E
    ci = jnp.where(routed, i, E - 1)

    @pl.when(i == 0)
    def _cast_x():
        xbf_ref[...] = x_ref[...].astype(jnp.bfloat16)

    x = xbf_ref[...]
    wu = jnp.where(routed, wu_ref[0], wus_ref[...]).astype(jnp.bfloat16)
    wd = jnp.where(routed, wd_ref[0], wds_ref[...]).astype(jnp.bfloat16)

    h = jnp.dot(x, wu, preferred_element_type=jnp.float32)
    h = jnp.square(jnp.maximum(h, 0.0)).astype(jnp.bfloat16)
    y = jnp.dot(h, wd, preferred_element_type=jnp.float32)

    lane = jax.lax.broadcasted_iota(jnp.int32, (T, E), 1)
    w = jnp.sum(jnp.where(lane == ci, cmb_ref[...], 0.0), axis=1, keepdims=True)
    w = jnp.where(routed, w, 1.0)

    @pl.when(i == 0)
    def _init():
        out_ref[...] = y * w

    @pl.when(i > 0)
    def _acc():
        out_ref[...] += y * w


def _build(interpret=False):
    router = pl.pallas_call(
        _router_kernel,
        grid=(T // TBLK,),
        in_specs=[
            pl.BlockSpec((TBLK, E), lambda t: (t, 0)),
            pl.BlockSpec((1, E), lambda t: (0, 0)),
        ],
        out_specs=pl.BlockSpec((TBLK, E), lambda t: (t, 0)),
        out_shape=jax.ShapeDtypeStruct((T, E), jnp.float32),
        interpret=interpret,
    )

    nsteps = E + IS // I
    moe = pl.pallas_call(
        _moe_kernel,
        grid=(nsteps,),
        in_specs=[
            pl.BlockSpec((T, E), lambda i: (0, 0)),
            pl.BlockSpec((T, H), lambda i: (0, 0)),
            pl.BlockSpec((1, H, I), lambda i: (jnp.where(i < E, i, E - 1), 0, 0)),
            pl.BlockSpec((1, I, H), lambda i: (jnp.where(i < E, i, E - 1), 0, 0)),
            pl.BlockSpec((H, I), lambda i: (0, jnp.where(i < E, 0, i - E))),
            pl.BlockSpec((I, H), lambda i: (jnp.where(i < E, 0, i - E), 0)),
        ],
        out_specs=pl.BlockSpec((T, H), lambda i: (0, 0)),
        out_shape=jax.ShapeDtypeStruct((T, H), jnp.float32),
        scratch_shapes=[pltpu.VMEM((T, H), jnp.bfloat16)],
        compiler_params=pltpu.CompilerParams(
            dimension_semantics=("arbitrary",),
        ),
        interpret=interpret,
    )
    return router, moe


@functools.partial(jax.jit, static_argnames=("interpret",))
def _run(hidden_states, Wg, e_bias, Wu, Wd, Wu_s, Wd_s, interpret=False):
    router, moe = _build(interpret)
    x = hidden_states.reshape(T, H)
    s = jax.nn.sigmoid(x.astype(jnp.float32) @ Wg.T)
    cmb = router(s, e_bias.reshape(1, E))
    out = moe(cmb, x, Wu, Wd, Wu_s, Wd_s)
    return out.reshape(hidden_states.shape)


def kernel(hidden_states, Wg, e_bias, Wu, Wd, Wu_s, Wd_s):
    return _run(hidden_states, Wg, e_bias, Wu, Wd, Wu_s, Wd_s)


# speedup vs baseline: 1.0020x; 1.0020x over previous
import functools

import jax
import jax.numpy as jnp
import numpy as np
from jax.experimental import pallas as pl
from jax.experimental.pallas import tpu as pltpu

H = 1024
E = 16
I = 512
IS = 2048
N_GROUP = 4
GROUP_SIZE = E // N_GROUP
TOPK_GROUP = 2
TOP_K = 8
ROUTED_SCALE = 2.5

T = 2048
TBLK = 256


def _rank_desc(v):
    B, N = v.shape
    idx = jax.lax.broadcasted_iota(jnp.int32, (B, N), 1)
    rank = jnp.zeros((B, N), jnp.float32)
    for j in range(N):
        colv = v[:, j:j + 1]
        beats = jnp.logical_or(colv > v,
                               jnp.logical_and(colv == v, j < idx))
        rank = rank + jnp.where(beats, 1.0, 0.0)
    return rank


def _router_kernel(s_ref, bias_ref, cmb_ref):
    s = s_ref[...]
    sc = s + bias_ref[...]

    col = [sc[:, j:j + 1] for j in range(E)]

    top2 = []
    for g in range(N_GROUP):
        a, b, c, d = col[4 * g], col[4 * g + 1], col[4 * g + 2], col[4 * g + 3]
        hi1, lo1 = jnp.maximum(a, b), jnp.minimum(a, b)
        hi2, lo2 = jnp.maximum(c, d), jnp.minimum(c, d)
        top2.append(jnp.maximum(jnp.maximum(hi1 + hi2, hi1 + lo1), hi2 + lo2))

    lane = jax.lax.broadcasted_iota(jnp.int32, (TBLK, E), 1)
    zero = jnp.zeros((TBLK, E), jnp.float32)
    esel = zero
    for g in range(N_GROUP):
        grank = 0
        for g2 in range(N_GROUP):
            if g2 == g:
                continue
            beats = jnp.logical_or(
                top2[g2] > top2[g],
                jnp.logical_and(top2[g2] == top2[g], g2 < g))
            grank = grank + jnp.where(beats, 1, 0)
        gsel = grank < TOPK_GROUP
        gmask = jnp.logical_and(lane >= 4 * g, lane < 4 * (g + 1))
        esel = esel + jnp.where(jnp.logical_and(gsel, gmask), 1.0, 0.0)

    scores_for_choice = jnp.where(esel > 0.5, sc, 0.0)

    erank = _rank_desc(scores_for_choice)
    sel = erank < TOP_K

    tw = jnp.where(sel, s, 0.0)
    denom = jnp.sum(tw, axis=1, keepdims=True) + 1e-20
    cmb_ref[...] = tw * (ROUTED_SCALE / denom)


def _moe_kernel(cmb_ref, x_ref, wu_ref, wd_ref, wus_ref, wds_ref, out_ref,
                xbf_ref):
    i = pl.program_id(0)
    routed = i < E
    ci = jnp.where(routed, i, E - 1)

    @pl.when(i == 0)
    def _cast_x():
        xbf_ref[...] = x_ref[...].astype(jnp.bfloat16)

    x = xbf_ref[...]
    wu = jnp.where(routed, wu_ref[0], wus_ref[...]).astype(jnp.bfloat16)
    wd = jnp.where(routed, wd_ref[0], wds_ref[...]).astype(jnp.bfloat16)

    lane = jax.lax.broadcasted_iota(jnp.int32, (T, E), 1)
    w = jnp.sum(jnp.where(lane == ci, cmb_ref[...], 0.0), axis=1, keepdims=True)
    w = jnp.where(routed, w, 1.0)

    h = jnp.dot(x, wu, preferred_element_type=jnp.float32)
    h = (jnp.square(jnp.maximum(h, 0.0)) * w).astype(jnp.bfloat16)

    @pl.when(i == 0)
    def _init():
        out_ref[...] = jnp.dot(h, wd, preferred_element_type=jnp.float32)

    @pl.when(i > 0)
    def _acc():
        out_ref[...] += jnp.dot(h, wd, preferred_element_type=jnp.float32)


def _build(interpret=False):
    router = pl.pallas_call(
        _router_kernel,
        grid=(T // TBLK,),
        in_specs=[
            pl.BlockSpec((TBLK, E), lambda t: (t, 0)),
            pl.BlockSpec((1, E), lambda t: (0, 0)),
        ],
        out_specs=pl.BlockSpec((TBLK, E), lambda t: (t, 0)),
        out_shape=jax.ShapeDtypeStruct((T, E), jnp.float32),
        interpret=interpret,
    )

    nsteps = E + IS // I
    moe = pl.pallas_call(
        _moe_kernel,
        grid=(nsteps,),
        in_specs=[
            pl.BlockSpec((T, E), lambda i: (0, 0)),
            pl.BlockSpec((T, H), lambda i: (0, 0)),
            pl.BlockSpec((1, H, I), lambda i: (jnp.where(i < E, i, E - 1), 0, 0)),
            pl.BlockSpec((1, I, H), lambda i: (jnp.where(i < E, i, E - 1), 0, 0)),
            pl.BlockSpec((H, I), lambda i: (0, jnp.where(i < E, 0, i - E))),
            pl.BlockSpec((I, H), lambda i: (jnp.where(i < E, 0, i - E), 0)),
        ],
        out_specs=pl.BlockSpec((T, H), lambda i: (0, 0)),
        out_shape=jax.ShapeDtypeStruct((T, H), jnp.float32),
        scratch_shapes=[pltpu.VMEM((T, H), jnp.bfloat16)],
        compiler_params=pltpu.CompilerParams(
            dimension_semantics=("arbitrary",),
        ),
        interpret=interpret,
    )
    return router, moe


@functools.partial(jax.jit, static_argnames=("interpret",))
def _run(hidden_states, Wg, e_bias, Wu, Wd, Wu_s, Wd_s, interpret=False):
    router, moe = _build(interpret)
    x = hidden_states.reshape(T, H)
    s = jax.nn.sigmoid(x.astype(jnp.float32) @ Wg.T)
    cmb = router(s, e_bias.reshape(1, E))
    out = moe(cmb, x, Wu, Wd, Wu_s, Wd_s)
    return out.reshape(hidden_states.shape)


def kernel(hidden_states, Wg, e_bias, Wu, Wd, Wu_s, Wd_s):
    return _run(hidden_states, Wg, e_bias, Wu, Wd, Wu_s, Wd_s)


# speedup vs baseline: 1.1046x; 1.1025x over previous
import functools

import jax
import jax.numpy as jnp
from jax.experimental import pallas as pl
from jax.experimental.pallas import tpu as pltpu

H = 1024
E = 16
I = 512
IS = 2048
N_GROUP = 4
GROUP_SIZE = E // N_GROUP
TOPK_GROUP = 2
TOP_K = 8
ROUTED_SCALE = 2.5

T = 2048


def _rank_desc(v):
    B, N = v.shape
    idx = jax.lax.broadcasted_iota(jnp.int32, (B, N), 1)
    rank = jnp.zeros((B, N), jnp.float32)
    for j in range(N):
        colv = v[:, j:j + 1]
        beats = jnp.logical_or(colv > v,
                               jnp.logical_and(colv == v, j < idx))
        rank = rank + jnp.where(beats, 1.0, 0.0)
    return rank


def _combine_from_scores(s, bias):
    B = s.shape[0]
    sc = s + bias

    col = [sc[:, j:j + 1] for j in range(E)]

    top2 = []
    for g in range(N_GROUP):
        a, b, c, d = col[4 * g], col[4 * g + 1], col[4 * g + 2], col[4 * g + 3]
        hi1, lo1 = jnp.maximum(a, b), jnp.minimum(a, b)
        hi2, lo2 = jnp.maximum(c, d), jnp.minimum(c, d)
        top2.append(jnp.maximum(jnp.maximum(hi1 + hi2, hi1 + lo1), hi2 + lo2))

    lane = jax.lax.broadcasted_iota(jnp.int32, (B, E), 1)
    esel = jnp.zeros((B, E), jnp.float32)
    for g in range(N_GROUP):
        grank = 0
        for g2 in range(N_GROUP):
            if g2 == g:
                continue
            beats = jnp.logical_or(
                top2[g2] > top2[g],
                jnp.logical_and(top2[g2] == top2[g], g2 < g))
            grank = grank + jnp.where(beats, 1, 0)
        gsel = grank < TOPK_GROUP
        gmask = jnp.logical_and(lane >= 4 * g, lane < 4 * (g + 1))
        esel = esel + jnp.where(jnp.logical_and(gsel, gmask), 1.0, 0.0)

    scores_for_choice = jnp.where(esel > 0.5, sc, 0.0)

    erank = _rank_desc(scores_for_choice)
    sel = erank < TOP_K

    tw = jnp.where(sel, s, 0.0)
    denom = jnp.sum(tw, axis=1, keepdims=True) + 1e-20
    return tw * (ROUTED_SCALE / denom)


def _moe_kernel(s_ref, bias_ref, x_ref, wu_ref, wd_ref, wus_ref, wds_ref,
                out_ref, xbf_ref, cmb_ref):
    i = pl.program_id(0)
    routed = i < E

    @pl.when(i == 0)
    def _prologue():
        xbf_ref[...] = x_ref[...].astype(jnp.bfloat16)
        cmb_ref[...] = _combine_from_scores(s_ref[...], bias_ref[...])

    x = xbf_ref[...]

    def expert_contrib(wu, wd, w):
        h = jnp.dot(x, wu, preferred_element_type=jnp.float32)
        h = (jnp.square(jnp.maximum(h, 0.0)) * w).astype(jnp.bfloat16)
        return jnp.dot(h, wd, preferred_element_type=jnp.float32)

    def routed_w():
        lane = jax.lax.broadcasted_iota(jnp.int32, (T, E), 1)
        return jnp.sum(jnp.where(lane == i, cmb_ref[...], 0.0),
                       axis=1, keepdims=True)

    @pl.when(i == 0)
    def _init():
        out_ref[...] = expert_contrib(
            wu_ref[0].astype(jnp.bfloat16), wd_ref[0].astype(jnp.bfloat16),
            routed_w())

    @pl.when(jnp.logical_and(i > 0, routed))
    def _expert():
        out_ref[...] += expert_contrib(
            wu_ref[0].astype(jnp.bfloat16), wd_ref[0].astype(jnp.bfloat16),
            routed_w())

    @pl.when(jnp.logical_not(routed))
    def _shared():
        out_ref[...] += expert_contrib(
            wus_ref[...].astype(jnp.bfloat16), wds_ref[...].astype(jnp.bfloat16),
            1.0)


def _build(interpret=False):
    nsteps = E + IS // I
    moe = pl.pallas_call(
        _moe_kernel,
        grid=(nsteps,),
        in_specs=[
            pl.BlockSpec((T, E), lambda i: (0, 0)),
            pl.BlockSpec((1, E), lambda i: (0, 0)),
            pl.BlockSpec((T, H), lambda i: (0, 0)),
            pl.BlockSpec((1, H, I), lambda i: (jnp.where(i < E, i, E - 1), 0, 0)),
            pl.BlockSpec((1, I, H), lambda i: (jnp.where(i < E, i, E - 1), 0, 0)),
            pl.BlockSpec((H, I), lambda i: (0, jnp.where(i < E, 0, i - E))),
            pl.BlockSpec((I, H), lambda i: (jnp.where(i < E, 0, i - E), 0)),
        ],
        out_specs=pl.BlockSpec((T, H), lambda i: (0, 0)),
        out_shape=jax.ShapeDtypeStruct((T, H), jnp.float32),
        scratch_shapes=[pltpu.VMEM((T, H), jnp.bfloat16),
                        pltpu.VMEM((T, E), jnp.float32)],
        compiler_params=pltpu.CompilerParams(
            dimension_semantics=("arbitrary",),
        ),
        interpret=interpret,
    )
    return moe


@functools.partial(jax.jit, static_argnames=("interpret",))
def _run(hidden_states, Wg, e_bias, Wu, Wd, Wu_s, Wd_s, interpret=False):
    moe = _build(interpret)
    x = hidden_states.reshape(T, H)
    s = jax.nn.sigmoid(x.astype(jnp.float32) @ Wg.T)
    out = moe(s, e_bias.reshape(1, E), x, Wu, Wd, Wu_s, Wd_s)
    return out.reshape(hidden_states.shape)


def kernel(hidden_states, Wg, e_bias, Wu, Wd, Wu_s, Wd_s):
    return _run(hidden_states, Wg, e_bias, Wu, Wd, Wu_s, Wd_s)


# speedup vs baseline: 1.1661x; 1.0557x over previous
import functools

import jax
import jax.numpy as jnp
from jax.experimental import pallas as pl
from jax.experimental.pallas import tpu as pltpu

H = 1024
E = 16
I = 512
IS = 2048
N_GROUP = 4
GROUP_SIZE = E // N_GROUP
TOPK_GROUP = 2
TOP_K = 8
ROUTED_SCALE = 2.5

T = 2048


def _rank_desc(v):
    B, N = v.shape
    idx = jax.lax.broadcasted_iota(jnp.int32, (B, N), 1)
    rank = jnp.zeros((B, N), jnp.float32)
    for j in range(N):
        colv = v[:, j:j + 1]
        beats = jnp.logical_or(colv > v,
                               jnp.logical_and(colv == v, j < idx))
        rank = rank + jnp.where(beats, 1.0, 0.0)
    return rank


def _combine_from_scores(s, bias):
    B = s.shape[0]
    sc = s + bias

    col = [sc[:, j:j + 1] for j in range(E)]

    top2 = []
    for g in range(N_GROUP):
        a, b, c, d = col[4 * g], col[4 * g + 1], col[4 * g + 2], col[4 * g + 3]
        hi1, lo1 = jnp.maximum(a, b), jnp.minimum(a, b)
        hi2, lo2 = jnp.maximum(c, d), jnp.minimum(c, d)
        top2.append(jnp.maximum(jnp.maximum(hi1 + hi2, hi1 + lo1), hi2 + lo2))

    lane = jax.lax.broadcasted_iota(jnp.int32, (B, E), 1)
    esel = jnp.zeros((B, E), jnp.float32)
    for g in range(N_GROUP):
        grank = 0
        for g2 in range(N_GROUP):
            if g2 == g:
                continue
            beats = jnp.logical_or(
                top2[g2] > top2[g],
                jnp.logical_and(top2[g2] == top2[g], g2 < g))
            grank = grank + jnp.where(beats, 1, 0)
        gsel = grank < TOPK_GROUP
        gmask = jnp.logical_and(lane >= 4 * g, lane < 4 * (g + 1))
        esel = esel + jnp.where(jnp.logical_and(gsel, gmask), 1.0, 0.0)

    scores_for_choice = jnp.where(esel > 0.5, sc, 0.0)

    erank = _rank_desc(scores_for_choice)
    sel = erank < TOP_K

    tw = jnp.where(sel, s, 0.0)
    denom = jnp.sum(tw, axis=1, keepdims=True) + 1e-20
    return tw * (ROUTED_SCALE / denom)


def _moe_kernel(s_ref, bias_ref, x_ref, wu_ref, wd_ref, wus_ref, wds_ref,
                out_ref, xbf_ref, cmb_ref):
    i = pl.program_id(0)
    routed = i < E

    @pl.when(i == 0)
    def _prologue():
        xbf_ref[...] = x_ref[...].astype(jnp.bfloat16)
        cmb_ref[...] = _combine_from_scores(s_ref[...], bias_ref[...])

    HALF = T // 2

    def expert_steps(wu, wd, w, init):
        for hf in range(2):
            r0 = hf * HALF
            xh = xbf_ref[r0:r0 + HALF, :]
            h = jnp.dot(xh, wu, preferred_element_type=jnp.float32)
            wh = w if isinstance(w, float) else w[r0:r0 + HALF, :]
            g = (jnp.square(jnp.maximum(h, 0.0)) * wh).astype(jnp.bfloat16)
            y = jnp.dot(g, wd, preferred_element_type=jnp.float32)
            if init:
                out_ref[r0:r0 + HALF, :] = y
            else:
                out_ref[r0:r0 + HALF, :] += y

    def routed_w():
        lane = jax.lax.broadcasted_iota(jnp.int32, (T, E), 1)
        return jnp.sum(jnp.where(lane == i, cmb_ref[...], 0.0),
                       axis=1, keepdims=True)

    @pl.when(i == 0)
    def _init():
        expert_steps(wu_ref[0].astype(jnp.bfloat16),
                     wd_ref[0].astype(jnp.bfloat16), routed_w(), True)

    @pl.when(jnp.logical_and(i > 0, routed))
    def _expert():
        expert_steps(wu_ref[0].astype(jnp.bfloat16),
                     wd_ref[0].astype(jnp.bfloat16), routed_w(), False)

    @pl.when(jnp.logical_not(routed))
    def _shared():
        expert_steps(wus_ref[...].astype(jnp.bfloat16),
                     wds_ref[...].astype(jnp.bfloat16), 1.0, False)


def _build(interpret=False):
    nsteps = E + IS // I
    moe = pl.pallas_call(
        _moe_kernel,
        grid=(nsteps,),
        in_specs=[
            pl.BlockSpec((T, E), lambda i: (0, 0)),
            pl.BlockSpec((1, E), lambda i: (0, 0)),
            pl.BlockSpec((T, H), lambda i: (0, 0)),
            pl.BlockSpec((1, H, I), lambda i: (jnp.where(i < E, i, E - 1), 0, 0)),
            pl.BlockSpec((1, I, H), lambda i: (jnp.where(i < E, i, E - 1), 0, 0)),
            pl.BlockSpec((H, I), lambda i: (0, jnp.where(i < E, 0, i - E))),
            pl.BlockSpec((I, H), lambda i: (jnp.where(i < E, 0, i - E), 0)),
        ],
        out_specs=pl.BlockSpec((T, H), lambda i: (0, 0)),
        out_shape=jax.ShapeDtypeStruct((T, H), jnp.float32),
        scratch_shapes=[pltpu.VMEM((T, H), jnp.bfloat16),
                        pltpu.VMEM((T, E), jnp.float32)],
        compiler_params=pltpu.CompilerParams(
            dimension_semantics=("arbitrary",),
        ),
        interpret=interpret,
    )
    return moe


@functools.partial(jax.jit, static_argnames=("interpret",))
def _run(hidden_states, Wg, e_bias, Wu, Wd, Wu_s, Wd_s, interpret=False):
    moe = _build(interpret)
    x = hidden_states.reshape(T, H)
    s = jax.nn.sigmoid(x.astype(jnp.float32) @ Wg.T)
    out = moe(s, e_bias.reshape(1, E), x, Wu, Wd, Wu_s, Wd_s)
    return out.reshape(hidden_states.shape)


def kernel(hidden_states, Wg, e_bias, Wu, Wd, Wu_s, Wd_s):
    return _run(hidden_states, Wg, e_bias, Wu, Wd, Wu_s, Wd_s)


# speedup vs baseline: 1.1756x; 1.0081x over previous
import functools

import jax
import jax.numpy as jnp
from jax.experimental import pallas as pl
from jax.experimental.pallas import tpu as pltpu

H = 1024
E = 16
I = 512
IS = 2048
N_GROUP = 4
GROUP_SIZE = E // N_GROUP
TOPK_GROUP = 2
TOP_K = 8
ROUTED_SCALE = 2.5

T = 2048


def _rank_desc(v):
    B, N = v.shape
    idx = jax.lax.broadcasted_iota(jnp.int32, (B, N), 1)
    rank = jnp.zeros((B, N), jnp.float32)
    for j in range(N):
        colv = v[:, j:j + 1]
        beats = jnp.logical_or(colv > v,
                               jnp.logical_and(colv == v, j < idx))
        rank = rank + jnp.where(beats, 1.0, 0.0)
    return rank


def _combine_from_scores(s, bias):
    B = s.shape[0]
    sc = s + bias

    col = [sc[:, j:j + 1] for j in range(E)]

    top2 = []
    for g in range(N_GROUP):
        a, b, c, d = col[4 * g], col[4 * g + 1], col[4 * g + 2], col[4 * g + 3]
        hi1, lo1 = jnp.maximum(a, b), jnp.minimum(a, b)
        hi2, lo2 = jnp.maximum(c, d), jnp.minimum(c, d)
        top2.append(jnp.maximum(jnp.maximum(hi1 + hi2, hi1 + lo1), hi2 + lo2))

    lane = jax.lax.broadcasted_iota(jnp.int32, (B, E), 1)
    esel = jnp.zeros((B, E), jnp.float32)
    for g in range(N_GROUP):
        grank = 0
        for g2 in range(N_GROUP):
            if g2 == g:
                continue
            beats = jnp.logical_or(
                top2[g2] > top2[g],
                jnp.logical_and(top2[g2] == top2[g], g2 < g))
            grank = grank + jnp.where(beats, 1, 0)
        gsel = grank < TOPK_GROUP
        gmask = jnp.logical_and(lane >= 4 * g, lane < 4 * (g + 1))
        esel = esel + jnp.where(jnp.logical_and(gsel, gmask), 1.0, 0.0)

    scores_for_choice = jnp.where(esel > 0.5, sc, 0.0)

    erank = _rank_desc(scores_for_choice)
    sel = erank < TOP_K

    tw = jnp.where(sel, s, 0.0)
    denom = jnp.sum(tw, axis=1, keepdims=True) + 1e-20
    return tw * (ROUTED_SCALE / denom)


def _moe_kernel(s_ref, bias_ref, x_ref, wu_ref, wd_ref, wus_ref, wds_ref,
                out_ref, xbf_ref, cmb_ref):
    i = pl.program_id(0)
    routed = i < E

    @pl.when(i == 0)
    def _prologue():
        xbf_ref[...] = x_ref[...].astype(jnp.bfloat16)
        cmb_ref[...] = _combine_from_scores(s_ref[...], bias_ref[...])

    HALF = T // 4

    def expert_steps(wu, wd, w, init):
        for hf in range(4):
            r0 = hf * HALF
            xh = xbf_ref[r0:r0 + HALF, :]
            h = jnp.dot(xh, wu, preferred_element_type=jnp.float32)
            wh = w if isinstance(w, float) else w[r0:r0 + HALF, :]
            g = (jnp.square(jnp.maximum(h, 0.0)) * wh).astype(jnp.bfloat16)
            y = jnp.dot(g, wd, preferred_element_type=jnp.float32)
            if init:
                out_ref[r0:r0 + HALF, :] = y
            else:
                out_ref[r0:r0 + HALF, :] += y

    def routed_w():
        lane = jax.lax.broadcasted_iota(jnp.int32, (T, E), 1)
        return jnp.sum(jnp.where(lane == i, cmb_ref[...], 0.0),
                       axis=1, keepdims=True)

    @pl.when(i == 0)
    def _init():
        expert_steps(wu_ref[0].astype(jnp.bfloat16),
                     wd_ref[0].astype(jnp.bfloat16), routed_w(), True)

    @pl.when(jnp.logical_and(i > 0, routed))
    def _expert():
        expert_steps(wu_ref[0].astype(jnp.bfloat16),
                     wd_ref[0].astype(jnp.bfloat16), routed_w(), False)

    @pl.when(jnp.logical_not(routed))
    def _shared():
        expert_steps(wus_ref[...].astype(jnp.bfloat16),
                     wds_ref[...].astype(jnp.bfloat16), 1.0, False)


def _build(interpret=False):
    nsteps = E + IS // I
    moe = pl.pallas_call(
        _moe_kernel,
        grid=(nsteps,),
        in_specs=[
            pl.BlockSpec((T, E), lambda i: (0, 0)),
            pl.BlockSpec((1, E), lambda i: (0, 0)),
            pl.BlockSpec((T, H), lambda i: (0, 0)),
            pl.BlockSpec((1, H, I), lambda i: (jnp.where(i < E, i, E - 1), 0, 0)),
            pl.BlockSpec((1, I, H), lambda i: (jnp.where(i < E, i, E - 1), 0, 0)),
            pl.BlockSpec((H, I), lambda i: (0, jnp.where(i < E, 0, i - E))),
            pl.BlockSpec((I, H), lambda i: (jnp.where(i < E, 0, i - E), 0)),
        ],
        out_specs=pl.BlockSpec((T, H), lambda i: (0, 0)),
        out_shape=jax.ShapeDtypeStruct((T, H), jnp.float32),
        scratch_shapes=[pltpu.VMEM((T, H), jnp.bfloat16),
                        pltpu.VMEM((T, E), jnp.float32)],
        compiler_params=pltpu.CompilerParams(
            dimension_semantics=("arbitrary",),
        ),
        interpret=interpret,
    )
    return moe


@functools.partial(jax.jit, static_argnames=("interpret",))
def _run(hidden_states, Wg, e_bias, Wu, Wd, Wu_s, Wd_s, interpret=False):
    moe = _build(interpret)
    x = hidden_states.reshape(T, H)
    s = jax.nn.sigmoid(x.astype(jnp.float32) @ Wg.T)
    out = moe(s, e_bias.reshape(1, E), x, Wu, Wd, Wu_s, Wd_s)
    return out.reshape(hidden_states.shape)


def kernel(hidden_states, Wg, e_bias, Wu, Wd, Wu_s, Wd_s):
    return _run(hidden_states, Wg, e_bias, Wu, Wd, Wu_s, Wd_s)


# speedup vs baseline: 1.1963x; 1.0176x over previous
import functools

import jax
import jax.numpy as jnp
from jax.experimental import pallas as pl
from jax.experimental.pallas import tpu as pltpu

H = 1024
E = 16
I = 512
IS = 2048
N_GROUP = 4
GROUP_SIZE = E // N_GROUP
TOPK_GROUP = 2
TOP_K = 8
ROUTED_SCALE = 2.5

T = 2048


def _rank_desc(v):
    B, N = v.shape
    idx = jax.lax.broadcasted_iota(jnp.int32, (B, N), 1)
    rank = jnp.zeros((B, N), jnp.float32)
    for j in range(N):
        colv = v[:, j:j + 1]
        beats = jnp.logical_or(colv > v,
                               jnp.logical_and(colv == v, j < idx))
        rank = rank + jnp.where(beats, 1.0, 0.0)
    return rank


def _combine_from_scores(s, bias):
    B = s.shape[0]
    sc = s + bias

    col = [sc[:, j:j + 1] for j in range(E)]

    top2 = []
    for g in range(N_GROUP):
        a, b, c, d = col[4 * g], col[4 * g + 1], col[4 * g + 2], col[4 * g + 3]
        hi1, lo1 = jnp.maximum(a, b), jnp.minimum(a, b)
        hi2, lo2 = jnp.maximum(c, d), jnp.minimum(c, d)
        top2.append(jnp.maximum(jnp.maximum(hi1 + hi2, hi1 + lo1), hi2 + lo2))

    lane = jax.lax.broadcasted_iota(jnp.int32, (B, E), 1)
    esel = jnp.zeros((B, E), jnp.float32)
    for g in range(N_GROUP):
        grank = 0
        for g2 in range(N_GROUP):
            if g2 == g:
                continue
            beats = jnp.logical_or(
                top2[g2] > top2[g],
                jnp.logical_and(top2[g2] == top2[g], g2 < g))
            grank = grank + jnp.where(beats, 1, 0)
        gsel = grank < TOPK_GROUP
        gmask = jnp.logical_and(lane >= 4 * g, lane < 4 * (g + 1))
        esel = esel + jnp.where(jnp.logical_and(gsel, gmask), 1.0, 0.0)

    scores_for_choice = jnp.where(esel > 0.5, sc, 0.0)

    erank = _rank_desc(scores_for_choice)
    sel = erank < TOP_K

    tw = jnp.where(sel, s, 0.0)
    denom = jnp.sum(tw, axis=1, keepdims=True) + 1e-20
    return tw * (ROUTED_SCALE / denom)


def _moe_kernel(s_ref, bias_ref, x_ref, wu_ref, wd_ref, wus_ref, wds_ref,
                out_ref, xbf_ref, cmb_ref):
    i = pl.program_id(0)
    routed = i < E // 2

    @pl.when(i == 0)
    def _prologue():
        xbf_ref[...] = x_ref[...].astype(jnp.bfloat16)
        cmb_ref[...] = _combine_from_scores(s_ref[...], bias_ref[...])

    CH = T // 4
    NPAIR = E // 2

    def routed_w(col):
        lane = jax.lax.broadcasted_iota(jnp.int32, (T, E), 1)
        return jnp.sum(jnp.where(lane == col, cmb_ref[...], 0.0),
                       axis=1, keepdims=True)

    def pair_body(init):
        wu0 = wu_ref[0].astype(jnp.bfloat16)
        wu1 = wu_ref[1].astype(jnp.bfloat16)
        wdc = jnp.concatenate([wd_ref[0], wd_ref[1]], axis=0).astype(jnp.bfloat16)
        w0 = routed_w(2 * i)
        w1 = routed_w(2 * i + 1)
        for hf in range(4):
            r0 = hf * CH
            xh = xbf_ref[r0:r0 + CH, :]
            h0 = jnp.dot(xh, wu0, preferred_element_type=jnp.float32)
            g0 = (jnp.square(jnp.maximum(h0, 0.0)) * w0[r0:r0 + CH, :]
                  ).astype(jnp.bfloat16)
            h1 = jnp.dot(xh, wu1, preferred_element_type=jnp.float32)
            g1 = (jnp.square(jnp.maximum(h1, 0.0)) * w1[r0:r0 + CH, :]
                  ).astype(jnp.bfloat16)
            g = jnp.concatenate([g0, g1], axis=1)
            y = jnp.dot(g, wdc, preferred_element_type=jnp.float32)
            if init:
                out_ref[r0:r0 + CH, :] = y
            else:
                out_ref[r0:r0 + CH, :] += y

    @pl.when(i == 0)
    def _pair0():
        pair_body(True)

    @pl.when(jnp.logical_and(i > 0, routed))
    def _pair():
        pair_body(False)

    @pl.when(jnp.logical_not(routed))
    def _shared():
        wus = wus_ref[...].astype(jnp.bfloat16)
        wds = wds_ref[...].astype(jnp.bfloat16)
        for hf in range(4):
            r0 = hf * CH
            xh = xbf_ref[r0:r0 + CH, :]
            h = jnp.dot(xh, wus, preferred_element_type=jnp.float32)
            g = jnp.square(jnp.maximum(h, 0.0)).astype(jnp.bfloat16)
            out_ref[r0:r0 + CH, :] += jnp.dot(
                g, wds, preferred_element_type=jnp.float32)


def _build(interpret=False):
    npair = E // 2
    nsteps = npair + IS // I
    moe = pl.pallas_call(
        _moe_kernel,
        grid=(nsteps,),
        in_specs=[
            pl.BlockSpec((T, E), lambda i: (0, 0)),
            pl.BlockSpec((1, E), lambda i: (0, 0)),
            pl.BlockSpec((T, H), lambda i: (0, 0)),
            pl.BlockSpec((2, H, I),
                         lambda i: (jnp.where(i < npair, i, npair - 1), 0, 0)),
            pl.BlockSpec((2, I, H),
                         lambda i: (jnp.where(i < npair, i, npair - 1), 0, 0)),
            pl.BlockSpec((H, I), lambda i: (0, jnp.where(i < npair, 0, i - npair))),
            pl.BlockSpec((I, H), lambda i: (jnp.where(i < npair, 0, i - npair), 0)),
        ],
        out_specs=pl.BlockSpec((T, H), lambda i: (0, 0)),
        out_shape=jax.ShapeDtypeStruct((T, H), jnp.float32),
        scratch_shapes=[pltpu.VMEM((T, H), jnp.bfloat16),
                        pltpu.VMEM((T, E), jnp.float32)],
        compiler_params=pltpu.CompilerParams(
            dimension_semantics=("arbitrary",),
        ),
        interpret=interpret,
    )
    return moe


@functools.partial(jax.jit, static_argnames=("interpret",))
def _run(hidden_states, Wg, e_bias, Wu, Wd, Wu_s, Wd_s, interpret=False):
    moe = _build(interpret)
    x = hidden_states.reshape(T, H)
    s = jax.nn.sigmoid(x.astype(jnp.float32) @ Wg.T)
    out = moe(s, e_bias.reshape(1, E), x, Wu, Wd, Wu_s, Wd_s)
    return out.reshape(hidden_states.shape)


def kernel(hidden_states, Wg, e_bias, Wu, Wd, Wu_s, Wd_s):
    return _run(hidden_states, Wg, e_bias, Wu, Wd, Wu_s, Wd_s)


# speedup vs baseline: 1.2003x; 1.0033x over previous
import functools

import jax
import jax.numpy as jnp
from jax.experimental import pallas as pl
from jax.experimental.pallas import tpu as pltpu

H = 1024
E = 16
I = 512
IS = 2048
N_GROUP = 4
GROUP_SIZE = E // N_GROUP
TOPK_GROUP = 2
TOP_K = 8
ROUTED_SCALE = 2.5

T = 2048


def _roll_l(v, k):
    k = k % E
    if k == 0:
        return v
    return jnp.concatenate([v[:, k:], v[:, :k]], axis=1)


def _combine_from_scores(s, bias):
    B = s.shape[0]
    sc = s + bias
    lane = jax.lax.broadcasted_iota(jnp.int32, (B, E), 1)

    def xor1(v):
        return jnp.where(lane % 2 == 0, _roll_l(v, 1), _roll_l(v, E - 1))

    def xor2(v):
        return jnp.where(lane % 4 < 2, _roll_l(v, 2), _roll_l(v, E - 2))

    p = xor1(sc)
    hi = jnp.maximum(sc, p)
    lo = jnp.minimum(sc, p)
    hi_o = xor2(hi)
    lo_o = xor2(lo)
    top2 = jnp.maximum(jnp.maximum(hi + hi_o, hi + lo), hi_o + lo_o)

    g_idx = lane // GROUP_SIZE
    grank = jnp.zeros((B, E), jnp.int32)
    for k in range(1, N_GROUP):
        gk = _roll_l(top2, 4 * k)
        gk_idx = (g_idx + k) % N_GROUP
        beats = jnp.logical_or(
            gk > top2, jnp.logical_and(gk == top2, gk_idx < g_idx))
        grank = grank + jnp.where(beats, 1, 0)
    gsel = grank < TOPK_GROUP

    scores_for_choice = jnp.where(gsel, sc, 0.0)

    v = scores_for_choice
    erank = jnp.zeros((B, E), jnp.int32)
    for k in range(1, E):
        vk = _roll_l(v, k)
        jk = (lane + k) % E
        beats = jnp.logical_or(
            vk > v, jnp.logical_and(vk == v, jk < lane))
        erank = erank + jnp.where(beats, 1, 0)
    sel = erank < TOP_K

    tw = jnp.where(sel, s, 0.0)
    denom = jnp.sum(tw, axis=1, keepdims=True) + 1e-20
    return tw * (ROUTED_SCALE / denom)


def _moe_kernel(s_ref, bias_ref, x_ref, wu_ref, wd_ref, wus_ref, wds_ref,
                out_ref, xbf_ref, cmb_ref):
    i = pl.program_id(0)
    routed = i < E // 2

    @pl.when(i == 0)
    def _prologue():
        xbf_ref[...] = x_ref[...].astype(jnp.bfloat16)
        cmb_ref[...] = _combine_from_scores(s_ref[...], bias_ref[...])

    CH = T // 4
    NPAIR = E // 2

    def routed_w(col):
        lane = jax.lax.broadcasted_iota(jnp.int32, (T, E), 1)
        return jnp.sum(jnp.where(lane == col, cmb_ref[...], 0.0),
                       axis=1, keepdims=True)

    def pair_body(init):
        wu0 = wu_ref[0].astype(jnp.bfloat16)
        wu1 = wu_ref[1].astype(jnp.bfloat16)
        wdc = jnp.concatenate([wd_ref[0], wd_ref[1]], axis=0).astype(jnp.bfloat16)
        w0 = routed_w(2 * i)
        w1 = routed_w(2 * i + 1)
        for hf in range(4):
            r0 = hf * CH
            xh = xbf_ref[r0:r0 + CH, :]
            h0 = jnp.dot(xh, wu0, preferred_element_type=jnp.float32)
            g0 = (jnp.square(jnp.maximum(h0, 0.0)) * w0[r0:r0 + CH, :]
                  ).astype(jnp.bfloat16)
            h1 = jnp.dot(xh, wu1, preferred_element_type=jnp.float32)
            g1 = (jnp.square(jnp.maximum(h1, 0.0)) * w1[r0:r0 + CH, :]
                  ).astype(jnp.bfloat16)
            g = jnp.concatenate([g0, g1], axis=1)
            y = jnp.dot(g, wdc, preferred_element_type=jnp.float32)
            if init:
                out_ref[r0:r0 + CH, :] = y
            else:
                out_ref[r0:r0 + CH, :] += y

    @pl.when(i == 0)
    def _pair0():
        pair_body(True)

    @pl.when(jnp.logical_and(i > 0, routed))
    def _pair():
        pair_body(False)

    @pl.when(jnp.logical_not(routed))
    def _shared():
        wus = wus_ref[...].astype(jnp.bfloat16)
        wds = wds_ref[...].astype(jnp.bfloat16)
        for hf in range(4):
            r0 = hf * CH
            xh = xbf_ref[r0:r0 + CH, :]
            h = jnp.dot(xh, wus, preferred_element_type=jnp.float32)
            g = jnp.square(jnp.maximum(h, 0.0)).astype(jnp.bfloat16)
            out_ref[r0:r0 + CH, :] += jnp.dot(
                g, wds, preferred_element_type=jnp.float32)


def _build(interpret=False):
    npair = E // 2
    nsteps = npair + IS // I
    moe = pl.pallas_call(
        _moe_kernel,
        grid=(nsteps,),
        in_specs=[
            pl.BlockSpec((T, E), lambda i: (0, 0)),
            pl.BlockSpec((1, E), lambda i: (0, 0)),
            pl.BlockSpec((T, H), lambda i: (0, 0)),
            pl.BlockSpec((2, H, I),
                         lambda i: (jnp.where(i < npair, i, npair - 1), 0, 0)),
            pl.BlockSpec((2, I, H),
                         lambda i: (jnp.where(i < npair, i, npair - 1), 0, 0)),
            pl.BlockSpec((H, I), lambda i: (0, jnp.where(i < npair, 0, i - npair))),
            pl.BlockSpec((I, H), lambda i: (jnp.where(i < npair, 0, i - npair), 0)),
        ],
        out_specs=pl.BlockSpec((T, H), lambda i: (0, 0)),
        out_shape=jax.ShapeDtypeStruct((T, H), jnp.float32),
        scratch_shapes=[pltpu.VMEM((T, H), jnp.bfloat16),
                        pltpu.VMEM((T, E), jnp.float32)],
        compiler_params=pltpu.CompilerParams(
            dimension_semantics=("arbitrary",),
        ),
        interpret=interpret,
    )
    return moe


@functools.partial(jax.jit, static_argnames=("interpret",))
def _run(hidden_states, Wg, e_bias, Wu, Wd, Wu_s, Wd_s, interpret=False):
    moe = _build(interpret)
    x = hidden_states.reshape(T, H)
    s = jax.nn.sigmoid(x.astype(jnp.float32) @ Wg.T)
    out = moe(s, e_bias.reshape(1, E), x, Wu, Wd, Wu_s, Wd_s)
    return out.reshape(hidden_states.shape)


def kernel(hidden_states, Wg, e_bias, Wu, Wd, Wu_s, Wd_s):
    return _run(hidden_states, Wg, e_bias, Wu, Wd, Wu_s, Wd_s)


# speedup vs baseline: 1.2395x; 1.0327x over previous
import functools

import jax
import jax.numpy as jnp
from jax.experimental import pallas as pl
from jax.experimental.pallas import tpu as pltpu

H = 1024
E = 16
I = 512
IS = 2048
N_GROUP = 4
GROUP_SIZE = E // N_GROUP
TOPK_GROUP = 2
TOP_K = 8
ROUTED_SCALE = 2.5

T = 2048


W128 = 128
TPR = W128 // E
BR = T * E // W128


def _roll128(v, k):
    k = k % W128
    if k == 0:
        return v
    return jnp.concatenate([v[:, k:], v[:, :k]], axis=1)


def _combine_from_scores(s, bias):
    lane = jax.lax.broadcasted_iota(jnp.int32, (BR, W128), 1)
    sub = lane % E

    def seg_rot(v, k):
        stay = (sub + k) < E
        return jnp.where(stay, _roll128(v, k), _roll128(v, k - E))

    sc = s + bias

    p = jnp.where(sub % 2 == 0, seg_rot(sc, 1), seg_rot(sc, E - 1))
    hi = jnp.maximum(sc, p)
    lo = jnp.minimum(sc, p)
    sw2 = lambda v: jnp.where(sub % 4 < 2, seg_rot(v, 2), seg_rot(v, E - 2))
    hi_o = sw2(hi)
    lo_o = sw2(lo)
    top2 = jnp.maximum(jnp.maximum(hi + hi_o, hi + lo), hi_o + lo_o)

    g_idx = sub // GROUP_SIZE
    grank = jnp.zeros((BR, W128), jnp.int32)
    for k in range(1, N_GROUP):
        gk = seg_rot(top2, GROUP_SIZE * k)
        gk_idx = (g_idx + k) % N_GROUP
        beats = jnp.logical_or(
            gk > top2, jnp.logical_and(gk == top2, gk_idx < g_idx))
        grank = grank + jnp.where(beats, 1, 0)
    gsel = grank < TOPK_GROUP

    scores_for_choice = jnp.where(gsel, sc, 0.0)

    v = scores_for_choice
    erank = jnp.zeros((BR, W128), jnp.int32)
    for k in range(1, E):
        vk = seg_rot(v, k)
        jk = (sub + k) % E
        beats = jnp.logical_or(
            vk > v, jnp.logical_and(vk == v, jk < sub))
        erank = erank + jnp.where(beats, 1, 0)
    sel = erank < TOP_K

    tw = jnp.where(sel, s, 0.0)
    denom = tw
    for k in (8, 4, 2, 1):
        denom = denom + seg_rot(denom, k)
    denom = denom + 1e-20
    return tw * (ROUTED_SCALE / denom)


def _router_kernel(s_ref, bias_ref, cmb_ref):
    cmb_ref[...] = _combine_from_scores(s_ref[...], bias_ref[...])


def _moe_kernel(cmb_ref, x_ref, wu_ref, wd_ref, wus_ref, wds_ref,
                out_ref, xbf_ref):
    i = pl.program_id(0)
    routed = i < E // 2

    @pl.when(i == 0)
    def _prologue():
        xbf_ref[...] = x_ref[...].astype(jnp.bfloat16)

    CH = T // 4
    NPAIR = E // 2

    def routed_w(col):
        lane = jax.lax.broadcasted_iota(jnp.int32, (T, E), 1)
        return jnp.sum(jnp.where(lane == col, cmb_ref[...], 0.0),
                       axis=1, keepdims=True)

    def pair_body(init):
        wu0 = wu_ref[0].astype(jnp.bfloat16)
        wu1 = wu_ref[1].astype(jnp.bfloat16)
        wdc = jnp.concatenate([wd_ref[0], wd_ref[1]], axis=0).astype(jnp.bfloat16)
        w0 = routed_w(2 * i)
        w1 = routed_w(2 * i + 1)
        for hf in range(4):
            r0 = hf * CH
            xh = xbf_ref[r0:r0 + CH, :]
            h0 = jnp.dot(xh, wu0, preferred_element_type=jnp.float32)
            g0 = (jnp.square(jnp.maximum(h0, 0.0)) * w0[r0:r0 + CH, :]
                  ).astype(jnp.bfloat16)
            h1 = jnp.dot(xh, wu1, preferred_element_type=jnp.float32)
            g1 = (jnp.square(jnp.maximum(h1, 0.0)) * w1[r0:r0 + CH, :]
                  ).astype(jnp.bfloat16)
            g = jnp.concatenate([g0, g1], axis=1)
            y = jnp.dot(g, wdc, preferred_element_type=jnp.float32)
            if init:
                out_ref[r0:r0 + CH, :] = y
            else:
                out_ref[r0:r0 + CH, :] += y

    @pl.when(i == 0)
    def _pair0():
        pair_body(True)

    @pl.when(jnp.logical_and(i > 0, routed))
    def _pair():
        pair_body(False)

    @pl.when(jnp.logical_not(routed))
    def _shared():
        wus = wus_ref[...].astype(jnp.bfloat16)
        wds = wds_ref[...].astype(jnp.bfloat16)
        for hf in range(4):
            r0 = hf * CH
            xh = xbf_ref[r0:r0 + CH, :]
            h = jnp.dot(xh, wus, preferred_element_type=jnp.float32)
            g = jnp.square(jnp.maximum(h, 0.0)).astype(jnp.bfloat16)
            out_ref[r0:r0 + CH, :] += jnp.dot(
                g, wds, preferred_element_type=jnp.float32)


def _build(interpret=False):
    npair = E // 2
    nsteps = npair + IS // I
    moe = pl.pallas_call(
        _moe_kernel,
        grid=(nsteps,),
        in_specs=[
            pl.BlockSpec((T, E), lambda i: (0, 0)),
            pl.BlockSpec((T, H), lambda i: (0, 0)),
            pl.BlockSpec((2, H, I),
                         lambda i: (jnp.where(i < npair, i, npair - 1), 0, 0)),
            pl.BlockSpec((2, I, H),
                         lambda i: (jnp.where(i < npair, i, npair - 1), 0, 0)),
            pl.BlockSpec((H, I), lambda i: (0, jnp.where(i < npair, 0, i - npair))),
            pl.BlockSpec((I, H), lambda i: (jnp.where(i < npair, 0, i - npair), 0)),
        ],
        out_specs=pl.BlockSpec((T, H), lambda i: (0, 0)),
        out_shape=jax.ShapeDtypeStruct((T, H), jnp.float32),
        scratch_shapes=[pltpu.VMEM((T, H), jnp.bfloat16)],
        compiler_params=pltpu.CompilerParams(
            dimension_semantics=("arbitrary",),
        ),
        interpret=interpret,
    )
    router = pl.pallas_call(
        _router_kernel,
        grid=(1,),
        in_specs=[
            pl.BlockSpec((BR, W128), lambda i: (0, 0)),
            pl.BlockSpec((1, W128), lambda i: (0, 0)),
        ],
        out_specs=pl.BlockSpec((BR, W128), lambda i: (0, 0)),
        out_shape=jax.ShapeDtypeStruct((BR, W128), jnp.float32),
        interpret=interpret,
    )
    return router, moe


@functools.partial(jax.jit, static_argnames=("interpret",))
def _run(hidden_states, Wg, e_bias, Wu, Wd, Wu_s, Wd_s, interpret=False):
    router, moe = _build(interpret)
    x = hidden_states.reshape(T, H)
    s = jax.nn.sigmoid(x.astype(jnp.float32) @ Wg.T)
    s128 = s.reshape(BR, W128)
    bias128 = jnp.tile(e_bias, TPR).reshape(1, W128)
    cmb = router(s128, bias128).reshape(T, E)
    out = moe(cmb, x, Wu, Wd, Wu_s, Wd_s)
    return out.reshape(hidden_states.shape)


def kernel(hidden_states, Wg, e_bias, Wu, Wd, Wu_s, Wd_s):
    return _run(hidden_states, Wg, e_bias, Wu, Wd, Wu_s, Wd_s)
